# baseline (speedup 1.0000x reference)
import functools

import jax
import jax.numpy as jnp
from jax import lax
from jax.experimental import pallas as pl
from jax.experimental.pallas import tpu as pltpu
from jax.experimental.pallas import tpu_sc as plsc

B = 16384
V = 1000
D = 128
H = 512

_NC = 2
_NS = 16
_NW = _NC * _NS
_CH = 128


def _sc_gather_kernel(nchunk, ut_hbm, it_hbm, uid_hbm, iid_hbm,
                      uout_hbm, iout_hbm,
                      uidx, iidx, ubuf0, ubuf1, ubuf2, ibuf0, ibuf1, ibuf2,
                      *sems):
    bpw = nchunk * _CH
    wid = lax.axis_index("s") * _NC + lax.axis_index("c")
    base = wid * bpw
    pltpu.sync_copy(uid_hbm.at[pl.ds(base, bpw)], uidx)
    pltpu.sync_copy(iid_hbm.at[pl.ds(base, bpw)], iidx)
    ubufs = (ubuf0, ubuf1, ubuf2)
    ibufs = (ibuf0, ibuf1, ibuf2)
    nbuf = 3
    g = [None] * nchunk
    w = [None] * nchunk
    for c in range(nchunk):
        s = c % nbuf
        if c >= nbuf:
            w[c - nbuf][0].wait()
            w[c - nbuf][1].wait()
        sl = pl.ds(c * _CH, _CH)
        g[c] = (pltpu.async_copy(ut_hbm.at[uidx.at[sl]], ubufs[s], sems[s]),
                pltpu.async_copy(it_hbm.at[iidx.at[sl]], ibufs[s], sems[nbuf + s]))
        if c >= 1:
            p = (c - 1) % nbuf
            osl = pl.ds(base + (c - 1) * _CH, _CH)
            g[c - 1][0].wait()
            g[c - 1][1].wait()
            w[c - 1] = (pltpu.async_copy(ubufs[p], uout_hbm.at[osl],
                                         sems[2 * nbuf + p]),
                        pltpu.async_copy(ibufs[p], iout_hbm.at[osl],
                                         sems[3 * nbuf + p]))
    c = nchunk - 1
    s = c % nbuf
    osl = pl.ds(base + c * _CH, _CH)
    g[c][0].wait()
    g[c][1].wait()
    w[c] = (pltpu.async_copy(ubufs[s], uout_hbm.at[osl], sems[2 * nbuf + s]),
            pltpu.async_copy(ibufs[s], iout_hbm.at[osl], sems[3 * nbuf + s]))
    for c in range(max(0, nchunk - nbuf), nchunk):
        w[c][0].wait()
        w[c][1].wait()


def _sc_gather(user_table, item_table, user_ids, item_ids):
    nb = user_ids.shape[0]
    bpw = nb // _NW
    nchunk = bpw // _CH
    mesh = plsc.VectorSubcoreMesh(core_axis_name="c", subcore_axis_name="s")
    f = pl.kernel(
        functools.partial(_sc_gather_kernel, nchunk),
        mesh=mesh,
        out_type=[
            jax.ShapeDtypeStruct((nb, D), jnp.float32),
            jax.ShapeDtypeStruct((nb, D), jnp.float32),
        ],
        scratch_types=[
            pltpu.VMEM((bpw,), jnp.int32),
            pltpu.VMEM((bpw,), jnp.int32),
            pltpu.VMEM((_CH, D), jnp.float32),
            pltpu.VMEM((_CH, D), jnp.float32),
            pltpu.VMEM((_CH, D), jnp.float32),
            pltpu.VMEM((_CH, D), jnp.float32),
            pltpu.VMEM((_CH, D), jnp.float32),
            pltpu.VMEM((_CH, D), jnp.float32),
        ] + [pltpu.SemaphoreType.DMA] * 12,
    )
    return f(user_table, item_table, user_ids, item_ids)


_BLK = 4096


def _tc_heads_kernel(u_ref, i_ref,
                     rw1, rb1, rw2, rb2,
                     dw1, db1, dw2, db2,
                     nw1, nb1, nw2, nb2,
                     ro, do, no):
    c = jnp.concatenate([u_ref[...], i_ref[...]], axis=1).astype(jnp.bfloat16)

    def head(w1, b1, w2, b2, o_ref):
        h = jnp.dot(c, w1[...], preferred_element_type=jnp.float32)
        h = jnp.maximum(h + b1[...], 0.0).astype(jnp.bfloat16)
        o = jax.lax.dot_general(w2[...], h, (((0,), (1,)), ((), ())),
                                preferred_element_type=jnp.float32)
        o_ref[...] = o.reshape(_BLK) + b2[0, 0]

    head(rw1, rb1, rw2, rb2, ro)
    head(dw1, db1, dw2, db2, do)
    head(nw1, nb1, nw2, nb2, no)


def _tc_heads(u_emb, i_emb, weights):
    nb = u_emb.shape[0]
    row_spec = pl.BlockSpec((_BLK, D), lambda i: (i, 0))
    w1_spec = pl.BlockSpec((2 * D, H), lambda i: (0, 0))
    b1_spec = pl.BlockSpec((1, H), lambda i: (0, 0))
    w2_spec = pl.BlockSpec((H, 1), lambda i: (0, 0))
    b2_spec = pl.BlockSpec((1, 1), lambda i: (0, 0))
    o_spec = pl.BlockSpec((_BLK,), lambda i: (i,))
    in_specs = [row_spec, row_spec]
    for _ in range(3):
        in_specs += [w1_spec, b1_spec, w2_spec, b2_spec]
    out_shape = [jax.ShapeDtypeStruct((nb,), jnp.float32)] * 3
    f = pl.pallas_call(
        _tc_heads_kernel,
        grid=(nb // _BLK,),
        in_specs=in_specs,
        out_specs=[o_spec] * 3,
        out_shape=out_shape,
    )
    return f(u_emb, i_emb, *weights)


_NSPLIT = 1


def kernel(user_ids, item_ids, user_table, item_table,
           rel_W1, rel_b1, rel_W2, rel_b2,
           div_W1, div_b1, div_W2, div_b2,
           nov_W1, nov_b1, nov_W2, nov_b2):
    weights = []
    for W1, b1, W2, b2 in ((rel_W1, rel_b1, rel_W2, rel_b2),
                           (div_W1, div_b1, div_W2, div_b2),
                           (nov_W1, nov_b1, nov_W2, nov_b2)):
        weights += [W1.astype(jnp.bfloat16), b1.reshape(1, H),
                    W2.astype(jnp.bfloat16), b2.reshape(1, 1)]

    nb = B // _NSPLIT
    embs = []
    for s in range(_NSPLIT):
        sl = slice(s * nb, (s + 1) * nb)
        embs.append(_sc_gather(user_table, item_table,
                               user_ids[sl], item_ids[sl]))
    outs = [_tc_heads(u, i, weights) for (u, i) in embs]

    rel, div, nov = (jnp.concatenate(parts) for parts in zip(*outs))
    return (rel.reshape(B, 1), div.reshape(B, 1), nov.reshape(B, 1))


# speedup vs baseline: 1.0136x; 1.0136x over previous
import functools

import jax
import jax.numpy as jnp
from jax import lax
from jax.experimental import pallas as pl
from jax.experimental.pallas import tpu as pltpu
from jax.experimental.pallas import tpu_sc as plsc

B = 16384
V = 1000
D = 128
H = 512

_NC = 2
_NS = 16
_NW = _NC * _NS
_CH = 128


def _sc_gather_kernel(nchunk, ut_hbm, it_hbm, uid_hbm, iid_hbm,
                      uout_hbm, iout_hbm,
                      uidx, iidx, ubuf0, ubuf1, ubuf2, ibuf0, ibuf1, ibuf2,
                      *sems):
    bpw = nchunk * _CH
    wid = lax.axis_index("s") * _NC + lax.axis_index("c")
    base = wid * bpw
    pltpu.sync_copy(uid_hbm.at[pl.ds(base, bpw)], uidx)
    pltpu.sync_copy(iid_hbm.at[pl.ds(base, bpw)], iidx)
    ubufs = (ubuf0, ubuf1, ubuf2)
    ibufs = (ibuf0, ibuf1, ibuf2)
    nbuf = 3
    g = [None] * nchunk
    w = [None] * nchunk
    for c in range(nchunk):
        s = c % nbuf
        if c >= nbuf:
            w[c - nbuf][0].wait()
            w[c - nbuf][1].wait()
        sl = pl.ds(c * _CH, _CH)
        g[c] = (pltpu.async_copy(ut_hbm.at[uidx.at[sl]], ubufs[s], sems[s]),
                pltpu.async_copy(it_hbm.at[iidx.at[sl]], ibufs[s], sems[nbuf + s]))
        if c >= 1:
            p = (c - 1) % nbuf
            osl = pl.ds(base + (c - 1) * _CH, _CH)
            g[c - 1][0].wait()
            g[c - 1][1].wait()
            w[c - 1] = (pltpu.async_copy(ubufs[p], uout_hbm.at[osl],
                                         sems[2 * nbuf + p]),
                        pltpu.async_copy(ibufs[p], iout_hbm.at[osl],
                                         sems[3 * nbuf + p]))
    c = nchunk - 1
    s = c % nbuf
    osl = pl.ds(base + c * _CH, _CH)
    g[c][0].wait()
    g[c][1].wait()
    w[c] = (pltpu.async_copy(ubufs[s], uout_hbm.at[osl], sems[2 * nbuf + s]),
            pltpu.async_copy(ibufs[s], iout_hbm.at[osl], sems[3 * nbuf + s]))
    for c in range(max(0, nchunk - nbuf), nchunk):
        w[c][0].wait()
        w[c][1].wait()


def _sc_gather(user_table, item_table, user_ids, item_ids):
    nb = user_ids.shape[0]
    bpw = nb // _NW
    nchunk = bpw // _CH
    mesh = plsc.VectorSubcoreMesh(core_axis_name="c", subcore_axis_name="s")
    f = pl.kernel(
        functools.partial(_sc_gather_kernel, nchunk),
        mesh=mesh,
        out_type=[
            jax.ShapeDtypeStruct((nb, D), jnp.float32),
            jax.ShapeDtypeStruct((nb, D), jnp.float32),
        ],
        scratch_types=[
            pltpu.VMEM((bpw,), jnp.int32),
            pltpu.VMEM((bpw,), jnp.int32),
            pltpu.VMEM((_CH, D), jnp.float32),
            pltpu.VMEM((_CH, D), jnp.float32),
            pltpu.VMEM((_CH, D), jnp.float32),
            pltpu.VMEM((_CH, D), jnp.float32),
            pltpu.VMEM((_CH, D), jnp.float32),
            pltpu.VMEM((_CH, D), jnp.float32),
        ] + [pltpu.SemaphoreType.DMA] * 12,
    )
    return f(user_table, item_table, user_ids, item_ids)


_BLK = 2048


def _tc_heads_kernel(u_ref, i_ref,
                     rw1, rb1, rw2, rb2,
                     dw1, db1, dw2, db2,
                     nw1, nb1, nw2, nb2,
                     ro, do, no):
    c = jnp.concatenate([u_ref[...], i_ref[...]], axis=1).astype(jnp.bfloat16)

    def head(w1, b1, w2, b2, o_ref):
        h = jnp.dot(c, w1[...], preferred_element_type=jnp.float32)
        h = jnp.maximum(h + b1[...], 0.0).astype(jnp.bfloat16)
        o = jax.lax.dot_general(w2[...], h, (((0,), (1,)), ((), ())),
                                preferred_element_type=jnp.float32)
        o_ref[...] = o.reshape(_BLK) + b2[0, 0]

    head(rw1, rb1, rw2, rb2, ro)
    head(dw1, db1, dw2, db2, do)
    head(nw1, nb1, nw2, nb2, no)


def _tc_heads(u_emb, i_emb, weights):
    nb = u_emb.shape[0]
    row_spec = pl.BlockSpec((_BLK, D), lambda i: (i, 0))
    w1_spec = pl.BlockSpec((2 * D, H), lambda i: (0, 0))
    b1_spec = pl.BlockSpec((1, H), lambda i: (0, 0))
    w2_spec = pl.BlockSpec((H, 1), lambda i: (0, 0))
    b2_spec = pl.BlockSpec((1, 1), lambda i: (0, 0))
    o_spec = pl.BlockSpec((_BLK,), lambda i: (i,))
    in_specs = [row_spec, row_spec]
    for _ in range(3):
        in_specs += [w1_spec, b1_spec, w2_spec, b2_spec]
    out_shape = [jax.ShapeDtypeStruct((nb,), jnp.float32)] * 3
    f = pl.pallas_call(
        _tc_heads_kernel,
        grid=(nb // _BLK,),
        in_specs=in_specs,
        out_specs=[o_spec] * 3,
        out_shape=out_shape,
    )
    return f(u_emb, i_emb, *weights)


_NSPLIT = 2


def kernel(user_ids, item_ids, user_table, item_table,
           rel_W1, rel_b1, rel_W2, rel_b2,
           div_W1, div_b1, div_W2, div_b2,
           nov_W1, nov_b1, nov_W2, nov_b2):
    weights = []
    for W1, b1, W2, b2 in ((rel_W1, rel_b1, rel_W2, rel_b2),
                           (div_W1, div_b1, div_W2, div_b2),
                           (nov_W1, nov_b1, nov_W2, nov_b2)):
        weights += [W1.astype(jnp.bfloat16), b1.reshape(1, H),
                    W2.astype(jnp.bfloat16), b2.reshape(1, 1)]

    nb = B // _NSPLIT
    embs = []
    for s in range(_NSPLIT):
        sl = slice(s * nb, (s + 1) * nb)
        embs.append(_sc_gather(user_table, item_table,
                               user_ids[sl], item_ids[sl]))
    outs = [_tc_heads(u, i, weights) for (u, i) in embs]

    rel, div, nov = (jnp.concatenate(parts) for parts in zip(*outs))
    return (rel.reshape(B, 1), div.reshape(B, 1), nov.reshape(B, 1))
